# baseline (speedup 1.0000x reference)
import functools

import jax
import jax.numpy as jnp
from jax import lax
from jax.experimental import pallas as pl
from jax.experimental.pallas import tpu as pltpu
from jax.experimental.pallas import tpu_sc as plsc

NC = 2
NS = 16
NW = NC * NS
BW = 128
LANES = 16


@functools.partial(jax.jit, static_argnums=(3, 4))
def _gather_rows_t(j_t, off_t, table2, n_l, d):
    assert n_l % 2 == 0 and n_l >= 6 and d % LANES == 0 and BW % LANES == 0

    @functools.partial(
        pl.kernel,
        mesh=plsc.VectorSubcoreMesh(
            core_axis_name="c", subcore_axis_name="s",
            num_cores=NC, num_subcores=NS,
        ),
        out_type=jax.ShapeDtypeStruct((n_l, d, NW * BW), jnp.float32),
        scratch_types=[
            pltpu.VMEM((n_l, BW), jnp.int32),
            pltpu.VMEM((n_l, BW), jnp.int32),
            pltpu.VMEM((2, BW, 2 * d), jnp.float32),
            pltpu.VMEM((2, d, BW), jnp.float32),
            pltpu.SemaphoreType.DMA,
            pltpu.SemaphoreType.DMA,
        ],
        compiler_params=pltpu.CompilerParams(needs_layout_passes=False),
    )
    def body(j_hbm, off_hbm, table_hbm, out_hbm, idx_v, off_v, g_v, t_v,
             gsem, ssem):
        wid = lax.axis_index("s") * NC + lax.axis_index("c")
        pltpu.sync_copy(j_hbm.at[wid], idx_v)
        pltpu.sync_copy(off_hbm.at[wid], off_v)
        bbase = wid * BW

        def g_copy(sl, l):
            return pltpu.make_async_copy(
                table_hbm.at[idx_v.at[l]], g_v.at[sl], gsem)

        def s_copy(sl, l):
            return pltpu.make_async_copy(
                t_v.at[sl], out_hbm.at[l, :, pl.ds(bbase, BW)], ssem)

        lane = lax.iota(jnp.int32, LANES)

        def transpose(sl, l):
            def tblk(blk, carry):
                bb = blk * LANES + lane
                offv = plsc.load_gather(off_v.at[l], [bb])
                bstart = blk * LANES
                for c in range(d):
                    vals = plsc.load_gather(g_v.at[sl], [bb, offv + c])
                    t_v[sl, c, pl.ds(bstart, LANES)] = vals
                return carry
            lax.fori_loop(0, BW // LANES, tblk, 0)

        def step(sl, l, first, last):
            g_copy(sl, l).wait()
            if not first:
                s_copy(sl, l - 2).wait()
            transpose(sl, l)
            if not last:
                g_copy(sl, l + 2).start()
            s_copy(sl, l).start()

        g_copy(0, 0).start()
        g_copy(1, 1).start()
        step(0, 0, True, False)
        step(1, 1, True, False)

        def mid(t, carry):
            step(0, 2 * t, False, False)
            step(1, 2 * t + 1, False, False)
            return carry
        lax.fori_loop(1, n_l // 2 - 1, mid, 0)

        step(0, n_l - 2, False, True)
        step(1, n_l - 1, False, True)
        s_copy(0, n_l - 2).wait()
        s_copy(1, n_l - 1).wait()

    return body(j_t, off_t, table2)


def kernel(ids, table):
    b, l = ids.shape
    v, d = table.shape
    assert b % (NW * BW) == 0 and (2 * d) % 128 == 0
    if v % 2:
        table = jnp.pad(table, ((0, 1), (0, 0)))
    table2 = table.reshape(-1, 2 * d)
    ids_t = (
        ids.astype(jnp.int32).reshape(NW, BW, l).transpose(0, 2, 1)
    )
    j_t = ids_t >> 1
    off_t = (ids_t & 1) * d
    out_t = _gather_rows_t(j_t, off_t, table2, l, d)
    return jnp.transpose(out_t, (2, 0, 1))


# speedup vs baseline: 1.6100x; 1.6100x over previous
import functools

import jax
import jax.numpy as jnp
from jax import lax
from jax.experimental import pallas as pl
from jax.experimental.pallas import tpu as pltpu
from jax.experimental.pallas import tpu_sc as plsc

NC = 2
NS = 16
NW = NC * NS
CHUNK = 128


@functools.partial(jax.jit, static_argnums=(2, 3))
def _gather_rows(ids_r, table, n_chunks, d):
    total = NW * n_chunks * CHUNK

    NBUF = 5
    LOOK = NBUF - 2
    assert n_chunks % NBUF == 0 and n_chunks // NBUF >= 3

    @functools.partial(
        pl.kernel,
        mesh=plsc.VectorSubcoreMesh(
            core_axis_name="c", subcore_axis_name="s",
            num_cores=NC, num_subcores=NS,
        ),
        out_type=jax.ShapeDtypeStruct((total, d), jnp.float32),
        scratch_types=[
            pltpu.VMEM((n_chunks, CHUNK), jnp.int32),
            pltpu.VMEM((NBUF, CHUNK, d), jnp.float32),
            pltpu.SemaphoreType.DMA,
            pltpu.SemaphoreType.DMA,
        ],
        compiler_params=pltpu.CompilerParams(use_tc_tiling_on_sc=False),
    )
    def body(ids_hbm, table_hbm, out_hbm, idx_v, rows_v, gsem, ssem):
        wid = lax.axis_index("s") * NC + lax.axis_index("c")
        pltpu.sync_copy(ids_hbm.at[wid], idx_v)
        base = wid * (n_chunks * CHUNK)

        def g_copy(slot, j):
            return pltpu.make_async_copy(
                table_hbm.at[idx_v.at[j]], rows_v.at[slot], gsem)

        def s_copy(slot, j):
            return pltpu.make_async_copy(
                rows_v.at[slot], out_hbm.at[pl.ds(base + j * CHUNK, CHUNK)],
                ssem)

        for b in range(LOOK):
            g_copy(b, b).start()

        def block(j0, first, last):
            for b in range(NBUF):
                j = j0 + b
                g_copy(b, j).wait()
                s_copy(b, j).start()
                if not (last and b >= NBUF - LOOK):
                    slot2 = (b + LOOK) % NBUF
                    if not (first and b < NBUF - LOOK):
                        s_copy(slot2, j - (NBUF - LOOK)).wait()
                    g_copy(slot2, j + LOOK).start()

        block(0, True, False)
        if n_chunks // NBUF > 2:
            def mid(t, carry):
                block(t * NBUF, False, False)
                return carry
            lax.fori_loop(1, n_chunks // NBUF - 1, mid, 0)
        block(n_chunks - NBUF, False, True)

        for j in range(n_chunks - NBUF, n_chunks):
            s_copy(j % NBUF, j).wait()

    return body(ids_r, table)


def kernel(ids, table):
    b, l = ids.shape
    v, d = table.shape
    total = b * l
    ids_flat = ids.reshape(-1).astype(jnp.int32)

    per_w = -(-total // NW)
    n_chunks = -(-per_w // CHUNK)
    padded = NW * n_chunks * CHUNK
    if padded != total:
        ids_flat = jnp.pad(ids_flat, (0, padded - total))
    ids_r = ids_flat.reshape(NW, n_chunks, CHUNK)

    out = _gather_rows(ids_r, table, n_chunks, d)
    return out[:total].reshape(b, l, d)
